# baseline (speedup 1.0000x reference)
import functools

import jax
import jax.numpy as jnp
from jax.experimental import pallas as pl
from jax.experimental.pallas import tpu as pltpu


def _softplus(x):
    return jnp.maximum(x, 0.0) + jnp.log1p(jnp.exp(-jnp.abs(x)))


def _encoder_kernel(x_ref, adj_ref, eps_ref,
                    w1_ref, b1_ref, wmu_ref, bmu_ref, wls_ref, bls_ref,
                    wd1_ref, bd1_ref,
                    hd_ref, kl_ref):
    Bb, N, F = x_ref.shape

    X = x_ref[...]
    A = adj_ref[...]
    eps = eps_ref[...]

    XW = jnp.dot(X.reshape(Bb * N, F), w1_ref[...],
                 preferred_element_type=jnp.float32)
    XW = XW.reshape(Bb, N, -1)
    AXW = jnp.einsum('bnk,bkh->bnh', A, XW,
                     preferred_element_type=jnp.float32)
    H = jnp.maximum(AXW + XW + b1_ref[...], 0.0)
    g = jnp.mean(H, axis=1)

    mu = jnp.dot(g, wmu_ref[...],
                 preferred_element_type=jnp.float32) + bmu_ref[...]
    log_std = jnp.dot(g, wls_ref[...],
                      preferred_element_type=jnp.float32) + bls_ref[...]
    std = jnp.exp(log_std)

    z = mu + std * eps
    kl = jnp.sum(0.5 * (z * z - eps * eps) - log_std, axis=-1)

    hd = jnp.maximum(
        jnp.dot(z, wd1_ref[...], preferred_element_type=jnp.float32)
        + bd1_ref[...], 0.0)

    hd_ref[...] = hd
    kl_ref[...] = kl.reshape(1, 1, Bb)


def _decoder_kernel(hd_ref, wd2_ref, bd2_ref, a_ref, re_ref):
    B, Nc, N = a_ref.shape
    hd = hd_ref[...]
    logits = jnp.dot(hd, wd2_ref[...],
                     preferred_element_type=jnp.float32) + bd2_ref[...]
    a = a_ref[...].reshape(B, Nc * N)
    lp = a * logits - _softplus(logits)
    re_ref[...] = jnp.sum(lp, axis=-1).reshape(1, 1, -1)


def _finalize_kernel(re_p_ref, kl_ref, out_ref, *, k, B):
    re_b = jnp.sum(re_p_ref[...], axis=0)
    xj = jnp.broadcast_to(re_b, (B, B))
    xi = jnp.broadcast_to(re_b.reshape(B, 1), (B, B))
    jj = jax.lax.broadcasted_iota(jnp.int32, (B, B), 1)
    ii = jax.lax.broadcasted_iota(jnp.int32, (B, B), 0)
    better = (xj > xi) | ((xj == xi) & (jj < ii))
    rank = jnp.sum(better.astype(jnp.float32), axis=1, keepdims=True)
    mask = (rank < k).astype(jnp.float32)
    RE = jnp.sum(re_b.reshape(B, 1) * mask) / k
    mean_kl = jnp.sum(kl_ref[...]) / B
    out_ref[...] = jnp.broadcast_to(-(RE - mean_kl), (1, 1))


def _pick_div(n, prefer):
    for d in prefer:
        if n % d == 0:
            return d
    return 1


def kernel(X, Adj, node_masks, eps,
           w1, b1, wmu, bmu, wls, bls, wd1, bd1, wd2, bd2):
    del node_masks
    B, N, F = X.shape
    eps = eps.reshape(B, -1)
    M = eps.shape[-1]
    Hd = w1.shape[-1]
    NN = N * N

    G1 = _pick_div(B, (2,))
    Bb = B // G1
    enc_params = (w1, b1, wmu, bmu, wls, bls, wd1, bd1)
    enc_in_specs = [
        pl.BlockSpec((Bb, N, F), lambda b: (b, 0, 0)),
        pl.BlockSpec((Bb, N, N), lambda b: (b, 0, 0)),
        pl.BlockSpec((Bb, M), lambda b: (b, 0)),
    ] + [pl.BlockSpec(p.shape, lambda b, _nd=p.ndim: (0,) * _nd)
         for p in enc_params]
    enc_out_specs = [
        pl.BlockSpec((Bb, Hd), lambda b: (b, 0)),
        pl.BlockSpec((1, 1, Bb), lambda b: (b, 0, 0)),
    ]
    enc_out_shape = [
        jax.ShapeDtypeStruct((B, Hd), jnp.float32),
        jax.ShapeDtypeStruct((G1, 1, Bb), jnp.float32),
    ]
    hd, kl_b = pl.pallas_call(
        _encoder_kernel,
        grid=(G1,),
        in_specs=enc_in_specs,
        out_specs=enc_out_specs,
        out_shape=enc_out_shape,
        compiler_params=pltpu.CompilerParams(
            dimension_semantics=("parallel",)),
    )(X, Adj, eps, *enc_params)

    Nc = _pick_div(N, (32, 16, 8, 4, 2))
    C = Nc * N
    G2 = N // Nc
    dec_in_specs = [
        pl.BlockSpec((B, Hd), lambda j: (0, 0)),
        pl.BlockSpec((Hd, C), lambda j: (0, j)),
        pl.BlockSpec((1, C), lambda j: (0, j)),
        pl.BlockSpec((B, Nc, N), lambda j: (0, j, 0)),
    ]
    re_p = pl.pallas_call(
        _decoder_kernel,
        grid=(G2,),
        in_specs=dec_in_specs,
        out_specs=pl.BlockSpec((1, 1, B), lambda j: (j, 0, 0)),
        out_shape=jax.ShapeDtypeStruct((G2, 1, B), jnp.float32),
        compiler_params=pltpu.CompilerParams(
            dimension_semantics=("parallel",)),
    )(hd, wd2, bd2, Adj)

    k = int(B * 0.05)
    re_b = jnp.sum(re_p.reshape(G2, B), axis=0)
    xj = re_b[None, :]
    xi = re_b[:, None]
    jj = jax.lax.broadcasted_iota(jnp.int32, (B, B), 1)
    ii = jax.lax.broadcasted_iota(jnp.int32, (B, B), 0)
    better = (xj > xi) | ((xj == xi) & (jj < ii))
    rank = jnp.sum(better.astype(jnp.float32), axis=1)
    RE = jnp.sum(re_b * (rank < k)) / k
    mean_kl = jnp.sum(kl_b) / B
    return -(RE - mean_kl)


# speedup vs baseline: 1.1334x; 1.1334x over previous
import functools

import jax
import jax.numpy as jnp
from jax.experimental import pallas as pl
from jax.experimental.pallas import tpu as pltpu


def _softplus(x):
    return jnp.maximum(x, 0.0) + jnp.log1p(jnp.exp(-jnp.abs(x)))


def _encoder_kernel(x_ref, adj_ref, eps_ref,
                    w1_ref, b1_ref, wmu_ref, bmu_ref, wls_ref, bls_ref,
                    wd1_ref, bd1_ref,
                    hd_ref, kl_ref):
    Bb, N, F = x_ref.shape

    X = x_ref[...]
    A = adj_ref[...]
    eps = eps_ref[...]

    XW = jnp.dot(X.reshape(Bb * N, F), w1_ref[...],
                 preferred_element_type=jnp.float32)
    XW = XW.reshape(Bb, N, -1)
    AXW = jnp.einsum('bnk,bkh->bnh', A, XW,
                     preferred_element_type=jnp.float32)
    H = jnp.maximum(AXW + XW + b1_ref[...], 0.0)
    g = jnp.mean(H, axis=1)

    mu = jnp.dot(g, wmu_ref[...],
                 preferred_element_type=jnp.float32) + bmu_ref[...]
    log_std = jnp.dot(g, wls_ref[...],
                      preferred_element_type=jnp.float32) + bls_ref[...]
    std = jnp.exp(log_std)

    z = mu + std * eps
    kl = jnp.sum(0.5 * (z * z - eps * eps) - log_std, axis=-1)

    hd = jnp.maximum(
        jnp.dot(z, wd1_ref[...], preferred_element_type=jnp.float32)
        + bd1_ref[...], 0.0)

    hd_ref[...] = hd
    kl_ref[...] = kl.reshape(1, 1, Bb)


def _decoder_kernel(hd_ref, wd2_ref, bd2_ref, a_ref, re_ref):
    B, Nc, N = a_ref.shape
    hd = hd_ref[...]
    logits = jnp.dot(hd, wd2_ref[...],
                     preferred_element_type=jnp.float32) + bd2_ref[...]
    a = a_ref[...].reshape(B, Nc * N)
    lp = a * logits - _softplus(logits)
    re_ref[...] = jnp.sum(lp, axis=-1).reshape(1, 1, -1)


def _finalize_kernel(re_p_ref, kl_ref, out_ref, *, k, B):
    re_b = jnp.sum(re_p_ref[...], axis=0)
    xj = jnp.broadcast_to(re_b, (B, B))
    xi = jnp.broadcast_to(re_b.reshape(B, 1), (B, B))
    jj = jax.lax.broadcasted_iota(jnp.int32, (B, B), 1)
    ii = jax.lax.broadcasted_iota(jnp.int32, (B, B), 0)
    better = (xj > xi) | ((xj == xi) & (jj < ii))
    rank = jnp.sum(better.astype(jnp.float32), axis=1, keepdims=True)
    mask = (rank < k).astype(jnp.float32)
    RE = jnp.sum(re_b.reshape(B, 1) * mask) / k
    mean_kl = jnp.sum(kl_ref[...]) / B
    out_ref[...] = jnp.broadcast_to(-(RE - mean_kl), (1, 1))


def _pick_div(n, prefer):
    for d in prefer:
        if n % d == 0:
            return d
    return 1


def kernel(X, Adj, node_masks, eps,
           w1, b1, wmu, bmu, wls, bls, wd1, bd1, wd2, bd2):
    del node_masks
    B, N, F = X.shape
    eps = eps.reshape(B, -1)
    M = eps.shape[-1]
    Hd = w1.shape[-1]
    NN = N * N

    G1 = _pick_div(B, (2,))
    Bb = B // G1
    enc_params = (w1, b1, wmu, bmu, wls, bls, wd1, bd1)
    enc_in_specs = [
        pl.BlockSpec((Bb, N, F), lambda b: (b, 0, 0)),
        pl.BlockSpec((Bb, N, N), lambda b: (b, 0, 0)),
        pl.BlockSpec((Bb, M), lambda b: (b, 0)),
    ] + [pl.BlockSpec(p.shape, lambda b, _nd=p.ndim: (0,) * _nd)
         for p in enc_params]
    enc_out_specs = [
        pl.BlockSpec((Bb, Hd), lambda b: (b, 0)),
        pl.BlockSpec((1, 1, Bb), lambda b: (b, 0, 0)),
    ]
    enc_out_shape = [
        jax.ShapeDtypeStruct((B, Hd), jnp.float32),
        jax.ShapeDtypeStruct((G1, 1, Bb), jnp.float32),
    ]
    hd, kl_b = pl.pallas_call(
        _encoder_kernel,
        grid=(G1,),
        in_specs=enc_in_specs,
        out_specs=enc_out_specs,
        out_shape=enc_out_shape,
        compiler_params=pltpu.CompilerParams(
            dimension_semantics=("parallel",)),
    )(X, Adj, eps, *enc_params)

    Nc = _pick_div(N, (32, 16, 8, 4, 2))
    C = Nc * N
    G2 = N // Nc
    dec_in_specs = [
        pl.BlockSpec((B, Hd), lambda j: (0, 0)),
        pl.BlockSpec((Hd, C), lambda j: (0, j)),
        pl.BlockSpec((1, C), lambda j: (0, j)),
        pl.BlockSpec((B, Nc, N), lambda j: (0, j, 0)),
    ]
    re_p = pl.pallas_call(
        _decoder_kernel,
        grid=(G2,),
        in_specs=dec_in_specs,
        out_specs=pl.BlockSpec((1, 1, B), lambda j: (j, 0, 0)),
        out_shape=jax.ShapeDtypeStruct((G2, 1, B), jnp.float32),
        compiler_params=pltpu.CompilerParams(
            dimension_semantics=("parallel",)),
    )(hd, wd2, bd2, Adj)

    k = int(B * 0.05)
    res = pl.pallas_call(
        functools.partial(_finalize_kernel, k=k, B=B),
        in_specs=[
            pl.BlockSpec((G2, 1, B), lambda: (0, 0, 0)),
            pl.BlockSpec((G1, 1, Bb), lambda: (0, 0, 0)),
        ],
        out_specs=pl.BlockSpec((1, 1), lambda: (0, 0)),
        out_shape=jax.ShapeDtypeStruct((1, 1), jnp.float32),
    )(re_p, kl_b)
    return res.reshape(())
